# Initial kernel scaffold; baseline (speedup 1.0000x reference)
import functools

import jax
import jax.numpy as jnp
from jax import lax
from jax.experimental import pallas as pl
from jax.experimental.pallas import tpu as pltpu, tpu_sc as plsc

_NS = 16
_L = 16


def _copy_call(mem2d, rows_blk):
  n_rows = mem2d.shape[0]
  grid = n_rows // rows_blk

  def body(x_ref, o_ref):
    o_ref[...] = x_ref[...]

  return pl.pallas_call(
      body,
      out_shape=jax.ShapeDtypeStruct(mem2d.shape, mem2d.dtype),
      grid=(grid,),
      in_specs=[pl.BlockSpec((rows_blk, mem2d.shape[1]), lambda i: (i, 0))],
      out_specs=pl.BlockSpec((rows_blk, mem2d.shape[1]), lambda i: (i, 0)),
  )(mem2d)


def _make_scatter(M, B, D):
  rows_per_tile = 128 // _NS
  row_w = B // 128
  chunk = rows_per_tile * row_w

  mesh = plsc.VectorSubcoreMesh(core_axis_name="c", subcore_axis_name="s")

  @functools.partial(
      pl.kernel,
      out_type=(),
      mesh=mesh,
      scratch_types=[
          pltpu.VMEM_SHARED((M + 512,), jnp.int32),
          pltpu.VMEM_SHARED((_NS, _L), jnp.int32),
          pltpu.VMEM((rows_per_tile, row_w), jnp.int32),
          pltpu.VMEM((rows_per_tile, row_w), jnp.int32),
          pltpu.VMEM((rows_per_tile, row_w), jnp.int32),
          pltpu.VMEM((rows_per_tile, row_w), jnp.int32),
          pltpu.VMEM((chunk, D), jnp.float32),
          pltpu.VMEM((_L,), jnp.int32),
          pltpu.VMEM((_NS, _L), jnp.int32),
          pltpu.SemaphoreType.DMA,
          pltpu.SemaphoreType.DMA,
      ],
  )
  def scatter(val_hbm, idx_hbm, iota_hbm, out_hbm,
              stamp, counts_sh, idxc, ivals, got, scat, rows, cvec, cbuf,
              sem, sem2):
    s = lax.axis_index("s")
    t0 = s * rows_per_tile
    lanes = lax.iota(jnp.int32, _L)
    dump = M + s * _L + lanes

    pltpu.sync_copy(idx_hbm.at[pl.ds(t0, rows_per_tile)], idxc)
    pltpu.sync_copy(iota_hbm.at[pl.ds(t0, rows_per_tile)], ivals)

    def scatter_round(src_idx):
      for j in range(rows_per_tile):
        pltpu.sync_copy(ivals.at[j], stamp.at[src_idx.at[j]])

    def gather_round():
      for j in range(rows_per_tile):
        pltpu.sync_copy(stamp.at[idxc.at[j]], got.at[j])

    def recompute_active():
      def grp(g, cnt):
        j = g // (row_w // _L)
        cpos = (g % (row_w // _L)) * _L
        iv = ivals[j, pl.ds(cpos, _L)]
        gv = got[j, pl.ds(cpos, _L)]
        xv = idxc[j, pl.ds(cpos, _L)]
        act = iv > gv
        scat[j, pl.ds(cpos, _L)] = jnp.where(act, xv, dump)
        ones = jnp.where(act, jnp.int32(1), jnp.int32(0))
        return cnt + jnp.sum(ones)
      return lax.fori_loop(0, chunk // _L, grp, jnp.int32(0))

    def exchange_counts(cnt):
      cvec[...] = jnp.full((_L,), cnt, jnp.int32)
      pltpu.sync_copy(cvec, counts_sh.at[s])
      plsc.subcore_barrier()
      pltpu.sync_copy(counts_sh, cbuf)
      acc = lax.fori_loop(
          0, _NS, lambda r, a: a + cbuf[r, :], jnp.zeros((_L,), jnp.int32))
      return jnp.sum(acc)

    scatter_round(idxc)
    plsc.subcore_barrier()
    gather_round()
    total = exchange_counts(recompute_active())

    def body(total):
      scatter_round(scat)
      plsc.subcore_barrier()
      gather_round()
      return exchange_counts(recompute_active())

    lax.while_loop(lambda t: t > 0, body, total)

    gets = [
        pltpu.async_copy(val_hbm.at[got.at[j]],
                         rows.at[pl.ds(j * row_w, row_w)], sem)
        for j in range(rows_per_tile)
    ]
    for g in gets:
      g.wait()
    puts = [
        pltpu.async_copy(rows.at[pl.ds(j * row_w, row_w)],
                         out_hbm.at[idxc.at[j]], sem2)
        for j in range(rows_per_tile)
    ]
    for p in puts:
      p.wait()

  return scatter


def kernel(mem, val, idx):
  M, D = mem.shape
  B = val.shape[0]

  mem2d = mem.reshape(M // 32, D * 32)
  out0 = _copy_call(mem2d, 250).reshape(M, D)

  idx2d = idx.reshape(128, B // 128)
  biota = jnp.arange(B, dtype=jnp.int32).reshape(128, B // 128)

  out_ref = jax.new_ref(out0)
  _make_scatter(M, B, D)(val, idx2d, biota, out_ref)
  return out_ref[...]


# speedup vs baseline: 1.8256x; 1.8256x over previous
import functools

import jax
import jax.numpy as jnp
from jax import lax
from jax.experimental import pallas as pl
from jax.experimental.pallas import tpu as pltpu, tpu_sc as plsc

_NS = 16
_L = 16


def _copy_call(mem2d, rows_blk):
  n_rows = mem2d.shape[0]
  grid = n_rows // rows_blk

  def body(x_ref, o_ref):
    o_ref[...] = x_ref[...]

  return pl.pallas_call(
      body,
      out_shape=jax.ShapeDtypeStruct(mem2d.shape, mem2d.dtype),
      grid=(grid,),
      in_specs=[pl.BlockSpec((rows_blk, mem2d.shape[1]), lambda i: (i, 0))],
      out_specs=pl.BlockSpec((rows_blk, mem2d.shape[1]), lambda i: (i, 0)),
  )(mem2d)


def _make_scatter(M, B, D):
  rows_per_tile = 128 // _NS
  row_w = B // 128
  chunk = rows_per_tile * row_w

  mesh = plsc.VectorSubcoreMesh(core_axis_name="c", subcore_axis_name="s")

  @functools.partial(
      pl.kernel,
      out_type=(),
      mesh=mesh,
      compiler_params=pltpu.CompilerParams(
          needs_layout_passes=False, use_tc_tiling_on_sc=False),
      scratch_types=[
          pltpu.VMEM_SHARED((M + 512,), jnp.int32),
          pltpu.VMEM_SHARED((_NS, _L), jnp.int32),
          pltpu.VMEM((rows_per_tile, row_w), jnp.int32),
          pltpu.VMEM((rows_per_tile, row_w), jnp.int32),
          pltpu.VMEM((rows_per_tile, row_w), jnp.int32),
          pltpu.VMEM((rows_per_tile, row_w), jnp.int32),
          pltpu.VMEM((chunk, D), jnp.float32),
          pltpu.VMEM((_L,), jnp.int32),
          pltpu.VMEM((_NS, _L), jnp.int32),
          pltpu.SemaphoreType.DMA,
          pltpu.SemaphoreType.DMA,
      ],
  )
  def scatter(val_hbm, idx_hbm, iota_hbm, out_hbm,
              stamp, counts_sh, idxc, ivals, got, scat, rows, cvec, cbuf,
              sem, sem2):
    s = lax.axis_index("s")
    t0 = s * rows_per_tile
    lanes = lax.iota(jnp.int32, _L)
    dump = M + s * _L + lanes

    pltpu.sync_copy(idx_hbm.at[pl.ds(t0, rows_per_tile)], idxc)
    pltpu.sync_copy(iota_hbm.at[pl.ds(t0, rows_per_tile)], ivals)

    def scatter_round(src_idx):
      for j in range(rows_per_tile):
        pltpu.sync_copy(ivals.at[j], stamp.at[src_idx.at[j]])

    def gather_round():
      for j in range(rows_per_tile):
        pltpu.sync_copy(stamp.at[idxc.at[j]], got.at[j])

    def recompute_active():
      def grp(g, cnt):
        j = g // (row_w // _L)
        cpos = (g % (row_w // _L)) * _L
        iv = ivals[j, pl.ds(cpos, _L)]
        gv = got[j, pl.ds(cpos, _L)]
        xv = idxc[j, pl.ds(cpos, _L)]
        act = iv > gv
        scat[j, pl.ds(cpos, _L)] = jnp.where(act, xv, dump)
        ones = jnp.where(act, jnp.int32(1), jnp.int32(0))
        return cnt + jnp.sum(ones)
      return lax.fori_loop(0, chunk // _L, grp, jnp.int32(0))

    def exchange_counts(cnt):
      cvec[...] = jnp.full((_L,), cnt, jnp.int32)
      pltpu.sync_copy(cvec, counts_sh.at[s])
      plsc.subcore_barrier()
      pltpu.sync_copy(counts_sh, cbuf)
      acc = lax.fori_loop(
          0, _NS, lambda r, a: a + cbuf[r, :], jnp.zeros((_L,), jnp.int32))
      return jnp.sum(acc)

    scatter_round(idxc)
    plsc.subcore_barrier()
    gather_round()
    total = exchange_counts(recompute_active())

    def body(total):
      scatter_round(scat)
      plsc.subcore_barrier()
      gather_round()
      return exchange_counts(recompute_active())

    lax.while_loop(lambda t: t > 0, body, total)

    gets = [
        pltpu.async_copy(val_hbm.at[got.at[j]],
                         rows.at[pl.ds(j * row_w, row_w)], sem)
        for j in range(rows_per_tile)
    ]
    for g in gets:
      g.wait()
    puts = [
        pltpu.async_copy(rows.at[pl.ds(j * row_w, row_w)],
                         out_hbm.at[idxc.at[j]], sem2)
        for j in range(rows_per_tile)
    ]
    for p in puts:
      p.wait()

  return scatter


def kernel(mem, val, idx):
  M, D = mem.shape
  B = val.shape[0]

  mem2d = mem.reshape(M * D // 128, 128)
  out0 = _copy_call(mem2d, 2000).reshape(M, D)

  idx2d = idx.reshape(128, B // 128)
  biota = jnp.arange(B, dtype=jnp.int32).reshape(128, B // 128)

  out_ref = jax.new_ref(out0)
  _make_scatter(M, B, D)(val, idx2d, biota, out_ref)
  return out_ref[...]


# speedup vs baseline: 6.7429x; 3.6936x over previous
import functools

import jax
import jax.numpy as jnp
from jax import lax
from jax.experimental import pallas as pl
from jax.experimental.pallas import tpu as pltpu, tpu_sc as plsc

_NS = 16
_L = 16
_G = 8192
_P = _G // 4
_NB = 123
_MAIN = 122 * _G
_WR = _NB * _P


def _transpose_in(memT, tail_arr):
  def body(x0, x1, x2, x3, t_ref, o_ref):
    i = pl.program_id(0)

    @pl.when(i < _NB - 1)
    def _():
      xcat = jnp.concatenate([x0[...], x1[...], x2[...], x3[...]], axis=0)
      o_ref[...] = xcat.T

    @pl.when(i == _NB - 1)
    def _():
      o_ref[...] = t_ref[...]

  in_spec = [
      pl.BlockSpec((32, _P), (lambda i, u=u: (0, jnp.minimum(4 * i + u, 487))))
      for u in range(4)
  ]
  return pl.pallas_call(
      body,
      out_shape=jax.ShapeDtypeStruct((_WR, 128), memT.dtype),
      grid=(_NB,),
      in_specs=in_spec + [pl.BlockSpec((_P, 128), lambda i: (0, 0))],
      out_specs=pl.BlockSpec((_P, 128), lambda i: (i, 0)),
  )(memT, memT, memT, memT, tail_arr)


def _transpose_out(rm, M):
  def body(x_ref, o_ref):
    xt = x_ref[...].T
    for u in range(4):
      o_ref[:, _P * u:_P * u + _P] = xt[32 * u:32 * u + 32, :]

  return pl.pallas_call(
      body,
      out_shape=jax.ShapeDtypeStruct((32, M), rm.dtype),
      grid=(_NB,),
      in_specs=[pl.BlockSpec((_P, 128), lambda i: (i, 0))],
      out_specs=pl.BlockSpec((32, _G), lambda i: (0, i)),
  )(rm)


def _make_winner(M, B, D):
  rows_per_tile = 128 // _NS
  row_w = B // 128
  chunk = rows_per_tile * row_w

  mesh = plsc.VectorSubcoreMesh(core_axis_name="c", subcore_axis_name="s")

  @functools.partial(
      pl.kernel,
      out_type=jax.ShapeDtypeStruct((B, D), jnp.float32),
      mesh=mesh,
      compiler_params=pltpu.CompilerParams(
          needs_layout_passes=False, use_tc_tiling_on_sc=False),
      scratch_types=[
          pltpu.VMEM_SHARED((M + 512,), jnp.int32),
          pltpu.VMEM_SHARED((_NS, _L), jnp.int32),
          pltpu.VMEM((rows_per_tile, row_w), jnp.int32),
          pltpu.VMEM((rows_per_tile, row_w), jnp.int32),
          pltpu.VMEM((rows_per_tile, row_w), jnp.int32),
          pltpu.VMEM((rows_per_tile, row_w), jnp.int32),
          pltpu.VMEM((chunk, D), jnp.float32),
          pltpu.VMEM((_L,), jnp.int32),
          pltpu.VMEM((_NS, _L), jnp.int32),
          pltpu.SemaphoreType.DMA,
      ],
  )
  def winner(val_hbm, idx_hbm, iota_hbm, rows_hbm,
             stamp, counts_sh, idxc, ivals, got, scat, rows, cvec, cbuf, sem):
    c = lax.axis_index("c")
    s = lax.axis_index("s")
    t0 = s * rows_per_tile
    lanes = lax.iota(jnp.int32, _L)
    dump = M + s * _L + lanes

    pltpu.sync_copy(idx_hbm.at[pl.ds(t0, rows_per_tile)], idxc)
    pltpu.sync_copy(iota_hbm.at[pl.ds(t0, rows_per_tile)], ivals)

    def scatter_round(src_idx):
      for j in range(rows_per_tile):
        pltpu.sync_copy(ivals.at[j], stamp.at[src_idx.at[j]])

    def gather_round():
      for j in range(rows_per_tile):
        pltpu.sync_copy(stamp.at[idxc.at[j]], got.at[j])

    def recompute_active():
      def grp(g, cnt):
        j = g // (row_w // _L)
        cpos = (g % (row_w // _L)) * _L
        iv = ivals[j, pl.ds(cpos, _L)]
        gv = got[j, pl.ds(cpos, _L)]
        xv = idxc[j, pl.ds(cpos, _L)]
        act = iv > gv
        scat[j, pl.ds(cpos, _L)] = jnp.where(act, xv, dump)
        ones = jnp.where(act, jnp.int32(1), jnp.int32(0))
        return cnt + jnp.sum(ones)
      return lax.fori_loop(0, chunk // _L, grp, jnp.int32(0))

    def exchange_counts(cnt):
      cvec[...] = jnp.full((_L,), cnt, jnp.int32)
      pltpu.sync_copy(cvec, counts_sh.at[s])
      plsc.subcore_barrier()
      pltpu.sync_copy(counts_sh, cbuf)
      acc = lax.fori_loop(
          0, _NS, lambda r, a: a + cbuf[r, :], jnp.zeros((_L,), jnp.int32))
      return jnp.sum(acc)

    scatter_round(idxc)
    plsc.subcore_barrier()
    gather_round()
    total = exchange_counts(recompute_active())

    def body(total):
      scatter_round(scat)
      plsc.subcore_barrier()
      gather_round()
      return exchange_counts(recompute_active())

    lax.while_loop(lambda t: t > 0, body, total)

    gets = [
        pltpu.async_copy(val_hbm.at[got.at[j]],
                         rows.at[pl.ds(j * row_w, row_w)], sem)
        for j in range(rows_per_tile)
    ]
    for g in gets:
      g.wait()

    @pl.when(c == 0)
    def _():
      pltpu.sync_copy(rows, rows_hbm.at[pl.ds(s * chunk, chunk)])

  return winner


def _make_apply(M, B, D):
  n_w = 2 * _NS
  rows_per_w = 128 // n_w
  row_w = B // 128
  chunk = rows_per_w * row_w

  mesh = plsc.VectorSubcoreMesh(core_axis_name="c", subcore_axis_name="s")

  @functools.partial(
      pl.kernel,
      out_type=(),
      mesh=mesh,
      compiler_params=pltpu.CompilerParams(
          needs_layout_passes=False, use_tc_tiling_on_sc=False),
      scratch_types=[
          pltpu.VMEM((rows_per_w, row_w), jnp.int32),
          pltpu.VMEM((rows_per_w, row_w), jnp.int32),
          pltpu.VMEM((chunk, D), jnp.float32),
          pltpu.SemaphoreType.DMA,
      ],
  )
  def apply(rows_hbm, idx_hbm, out_hbm, idxc, q_idx, rows, sem):
    c = lax.axis_index("c")
    s = lax.axis_index("s")
    wid = s * 2 + c
    t0 = wid * rows_per_w

    pltpu.sync_copy(idx_hbm.at[pl.ds(t0, rows_per_w)], idxc)
    pltpu.sync_copy(rows_hbm.at[pl.ds(wid * chunk, chunk)], rows)

    def grp(g, _):
      j = g // (row_w // _L)
      cpos = (g % (row_w // _L)) * _L
      r = idxc[j, pl.ds(cpos, _L)]
      rg = r % _G
      q = 4 * (_P * (r // _G) + r % _P) + rg // _P
      q_idx[j, pl.ds(cpos, _L)] = q
      return 0
    lax.fori_loop(0, chunk // _L, grp, 0)

    puts = [
        pltpu.async_copy(rows.at[pl.ds(j * row_w, row_w)],
                         out_hbm.at[q_idx.at[j]], sem)
        for j in range(rows_per_w)
    ]
    for p in puts:
      p.wait()

  return apply


def kernel(mem, val, idx):
  M, D = mem.shape
  B = val.shape[0]

  idx2d = idx.reshape(128, B // 128)
  biota = jnp.arange(B, dtype=jnp.int32).reshape(128, B // 128)

  rows = _make_winner(M, B, D)(val, idx2d, biota)

  memT = mem.T
  tail_arr = jnp.pad(mem[_MAIN:], ((0, _P - (M - _MAIN)), (0, 128 - D)))
  rm = _transpose_in(memT, tail_arr)

  out_ref = jax.new_ref(rm.reshape(_WR * 4, D))
  _make_apply(M, B, D)(rows, idx2d, out_ref)

  outP = _transpose_out(out_ref[...].reshape(_WR, 128), M)
  return outP.T


# speedup vs baseline: 8.7960x; 1.3045x over previous
import functools

import jax
import jax.numpy as jnp
from jax import lax
from jax.experimental import pallas as pl
from jax.experimental.pallas import tpu as pltpu, tpu_sc as plsc

_NS = 16
_L = 16
_G = 16384
_P = _G // 4
_NB = 62
_MAIN = (_NB - 1) * _G
_WR = _NB * _P


def _transpose_in(memT, tail_arr):
  def body(x0, x1, x2, x3, t_ref, o_ref):
    i = pl.program_id(0)

    @pl.when(i < _NB - 1)
    def _():
      xcat = jnp.concatenate([x0[...], x1[...], x2[...], x3[...]], axis=0)
      o_ref[...] = xcat.T

    @pl.when(i == _NB - 1)
    def _():
      o_ref[...] = t_ref[...]

  in_spec = [
      pl.BlockSpec((32, _P),
                   (lambda i, u=u: (0, jnp.minimum(4 * i + u,
                                                   4 * (_NB - 1) - 1))))
      for u in range(4)
  ]
  return pl.pallas_call(
      body,
      out_shape=jax.ShapeDtypeStruct((_WR, 128), memT.dtype),
      grid=(_NB,),
      in_specs=in_spec + [pl.BlockSpec((_P, 128), lambda i: (0, 0))],
      out_specs=pl.BlockSpec((_P, 128), lambda i: (i, 0)),
  )(memT, memT, memT, memT, tail_arr)


def _transpose_out(rm, M):
  def body(x_ref, o_ref):
    xt = x_ref[...].T
    for u in range(4):
      o_ref[:, _P * u:_P * u + _P] = xt[32 * u:32 * u + 32, :]

  return pl.pallas_call(
      body,
      out_shape=jax.ShapeDtypeStruct((32, M), rm.dtype),
      grid=(_NB,),
      in_specs=[pl.BlockSpec((_P, 128), lambda i: (i, 0))],
      out_specs=pl.BlockSpec((32, _G), lambda i: (0, i)),
  )(rm)


def _make_winner(M, B, D):
  rows_per_tile = 128 // _NS
  row_w = B // 128
  chunk = rows_per_tile * row_w

  mesh = plsc.VectorSubcoreMesh(core_axis_name="c", subcore_axis_name="s")

  @functools.partial(
      pl.kernel,
      out_type=jax.ShapeDtypeStruct((B, D), jnp.float32),
      mesh=mesh,
      compiler_params=pltpu.CompilerParams(
          needs_layout_passes=False, use_tc_tiling_on_sc=False),
      scratch_types=[
          pltpu.VMEM_SHARED((M + 512,), jnp.int32),
          pltpu.VMEM_SHARED((_NS, _L), jnp.int32),
          pltpu.VMEM((rows_per_tile, row_w), jnp.int32),
          pltpu.VMEM((rows_per_tile, row_w), jnp.int32),
          pltpu.VMEM((rows_per_tile, row_w), jnp.int32),
          pltpu.VMEM((rows_per_tile, row_w), jnp.int32),
          pltpu.VMEM((chunk, D), jnp.float32),
          pltpu.VMEM((_L,), jnp.int32),
          pltpu.VMEM((_NS, _L), jnp.int32),
          pltpu.SemaphoreType.DMA,
      ],
  )
  def winner(val_hbm, idx_hbm, iota_hbm, rows_hbm,
             stamp, counts_sh, idxc, ivals, got, scat, rows, cvec, cbuf, sem):
    c = lax.axis_index("c")
    s = lax.axis_index("s")
    t0 = s * rows_per_tile
    lanes = lax.iota(jnp.int32, _L)
    dump = M + s * _L + lanes

    pltpu.sync_copy(idx_hbm.at[pl.ds(t0, rows_per_tile)], idxc)
    pltpu.sync_copy(iota_hbm.at[pl.ds(t0, rows_per_tile)], ivals)

    def scatter_round(src_idx):
      for j in range(rows_per_tile):
        pltpu.sync_copy(ivals.at[j], stamp.at[src_idx.at[j]])

    def gather_round():
      for j in range(rows_per_tile):
        pltpu.sync_copy(stamp.at[idxc.at[j]], got.at[j])

    def recompute_active():
      def grp(g, cnt):
        j = g // (row_w // _L)
        cpos = (g % (row_w // _L)) * _L
        iv = ivals[j, pl.ds(cpos, _L)]
        gv = got[j, pl.ds(cpos, _L)]
        xv = idxc[j, pl.ds(cpos, _L)]
        act = iv > gv
        scat[j, pl.ds(cpos, _L)] = jnp.where(act, xv, dump)
        ones = jnp.where(act, jnp.int32(1), jnp.int32(0))
        return cnt + jnp.sum(ones)
      return lax.fori_loop(0, chunk // _L, grp, jnp.int32(0))

    def exchange_counts(cnt):
      cvec[...] = jnp.full((_L,), cnt, jnp.int32)
      pltpu.sync_copy(cvec, counts_sh.at[s])
      plsc.subcore_barrier()
      pltpu.sync_copy(counts_sh, cbuf)
      acc = lax.fori_loop(
          0, _NS, lambda r, a: a + cbuf[r, :], jnp.zeros((_L,), jnp.int32))
      return jnp.sum(acc)

    scatter_round(idxc)
    plsc.subcore_barrier()
    gather_round()
    total = exchange_counts(recompute_active())

    def body(total):
      scatter_round(scat)
      plsc.subcore_barrier()
      gather_round()
      return exchange_counts(recompute_active())

    lax.while_loop(lambda t: t > 0, body, total)

    gets = [
        pltpu.async_copy(val_hbm.at[got.at[j]],
                         rows.at[pl.ds(j * row_w, row_w)], sem)
        for j in range(rows_per_tile)
    ]
    for g in gets:
      g.wait()

    @pl.when(c == 0)
    def _():
      pltpu.sync_copy(rows, rows_hbm.at[pl.ds(s * chunk, chunk)])

  return winner


def _make_apply(M, B, D):
  n_w = 2 * _NS
  rows_per_w = 128 // n_w
  row_w = B // 128
  chunk = rows_per_w * row_w

  mesh = plsc.VectorSubcoreMesh(core_axis_name="c", subcore_axis_name="s")

  @functools.partial(
      pl.kernel,
      out_type=(),
      mesh=mesh,
      compiler_params=pltpu.CompilerParams(
          needs_layout_passes=False, use_tc_tiling_on_sc=False),
      scratch_types=[
          pltpu.VMEM((rows_per_w, row_w), jnp.int32),
          pltpu.VMEM((rows_per_w, row_w), jnp.int32),
          pltpu.VMEM((chunk, D), jnp.float32),
          pltpu.SemaphoreType.DMA,
      ],
  )
  def apply(rows_hbm, idx_hbm, out_hbm, idxc, q_idx, rows, sem):
    c = lax.axis_index("c")
    s = lax.axis_index("s")
    wid = s * 2 + c
    t0 = wid * rows_per_w

    pltpu.sync_copy(idx_hbm.at[pl.ds(t0, rows_per_w)], idxc)
    pltpu.sync_copy(rows_hbm.at[pl.ds(wid * chunk, chunk)], rows)

    def grp(g, _):
      j = g // (row_w // _L)
      cpos = (g % (row_w // _L)) * _L
      r = idxc[j, pl.ds(cpos, _L)]
      rg = r % _G
      q = 4 * (_P * (r // _G) + r % _P) + rg // _P
      q_idx[j, pl.ds(cpos, _L)] = q
      return 0
    lax.fori_loop(0, chunk // _L, grp, 0)

    puts = [
        pltpu.async_copy(rows.at[pl.ds(j * row_w, row_w)],
                         out_hbm.at[q_idx.at[j]], sem)
        for j in range(rows_per_w)
    ]
    for p in puts:
      p.wait()

  return apply


def kernel(mem, val, idx):
  M, D = mem.shape
  B = val.shape[0]

  idx2d = idx.reshape(128, B // 128)
  biota = jnp.arange(B, dtype=jnp.int32).reshape(128, B // 128)

  rows = _make_winner(M, B, D)(val, idx2d, biota)

  memT = mem.T
  tail_arr = jnp.pad(mem[_MAIN:], ((0, _P - (M - _MAIN)), (0, 128 - D)))
  rm = _transpose_in(memT, tail_arr)

  out_ref = jax.new_ref(rm.reshape(_WR * 4, D))
  _make_apply(M, B, D)(rows, idx2d, out_ref)

  outP = _transpose_out(out_ref[...].reshape(_WR, 128), M)
  return outP.T


# speedup vs baseline: 9.9478x; 1.1310x over previous
import functools

import jax
import jax.numpy as jnp
from jax import lax
from jax.experimental import pallas as pl
from jax.experimental.pallas import tpu as pltpu, tpu_sc as plsc

_NS = 16
_L = 16
_G = 32768
_P = _G // 4
_NB = 31
_TAIL0 = 999424
_NFULL_LAST = _TAIL0 // _P - 4 * (_NB - 1)
_CLAMP = _TAIL0 // _P - 1
_WR = _NB * _P


def _transpose_in(memT, tailT):
  def body(x0, x1, x2, x3, t_ref, o_ref):
    i = pl.program_id(0)
    xs = (x0, x1, x2, x3)

    @pl.when(i < _NB - 1)
    def _():
      xcat = jnp.concatenate([x[...] for x in xs], axis=0)
      o_ref[...] = xcat.T

    @pl.when(i == _NB - 1)
    def _():
      pieces = [xs[u][...] if u < _NFULL_LAST else t_ref[...]
                for u in range(4)]
      o_ref[...] = jnp.concatenate(pieces, axis=0).T

  in_spec = [
      pl.BlockSpec((32, _P),
                   (lambda i, u=u: (0, jnp.minimum(4 * i + u, _CLAMP))))
      for u in range(4)
  ]
  return pl.pallas_call(
      body,
      out_shape=jax.ShapeDtypeStruct((_WR, 128), memT.dtype),
      grid=(_NB,),
      in_specs=in_spec + [pl.BlockSpec((32, _P), lambda i: (0, 0))],
      out_specs=pl.BlockSpec((_P, 128), lambda i: (i, 0)),
  )(memT, memT, memT, memT, tailT)


def _transpose_out(rm, M):
  def body(x_ref, o_ref):
    xt = x_ref[...].T
    for u in range(4):
      o_ref[:, _P * u:_P * u + _P] = xt[32 * u:32 * u + 32, :]

  return pl.pallas_call(
      body,
      out_shape=jax.ShapeDtypeStruct((32, M), rm.dtype),
      grid=(_NB,),
      in_specs=[pl.BlockSpec((_P, 128), lambda i: (i, 0))],
      out_specs=pl.BlockSpec((32, _G), lambda i: (0, i)),
  )(rm)


def _make_winner(M, B, D):
  rows_per_tile = 128 // _NS
  row_w = B // 128
  chunk = rows_per_tile * row_w

  mesh = plsc.VectorSubcoreMesh(core_axis_name="c", subcore_axis_name="s")

  @functools.partial(
      pl.kernel,
      out_type=jax.ShapeDtypeStruct((B, D), jnp.float32),
      mesh=mesh,
      compiler_params=pltpu.CompilerParams(
          needs_layout_passes=False, use_tc_tiling_on_sc=False),
      scratch_types=[
          pltpu.VMEM_SHARED((M + 512,), jnp.int32),
          pltpu.VMEM_SHARED((_NS, _L), jnp.int32),
          pltpu.VMEM((rows_per_tile, row_w), jnp.int32),
          pltpu.VMEM((rows_per_tile, row_w), jnp.int32),
          pltpu.VMEM((rows_per_tile, row_w), jnp.int32),
          pltpu.VMEM((rows_per_tile, row_w), jnp.int32),
          pltpu.VMEM((chunk, D), jnp.float32),
          pltpu.VMEM((_L,), jnp.int32),
          pltpu.VMEM((_NS, _L), jnp.int32),
          pltpu.SemaphoreType.DMA,
      ],
  )
  def winner(val_hbm, idx_hbm, iota_hbm, rows_hbm,
             stamp, counts_sh, idxc, ivals, got, scat, rows, cvec, cbuf, sem):
    c = lax.axis_index("c")
    s = lax.axis_index("s")
    t0 = s * rows_per_tile
    lanes = lax.iota(jnp.int32, _L)
    dump = M + s * _L + lanes

    pltpu.sync_copy(idx_hbm.at[pl.ds(t0, rows_per_tile)], idxc)
    pltpu.sync_copy(iota_hbm.at[pl.ds(t0, rows_per_tile)], ivals)

    def scatter_round(src_idx):
      for j in range(rows_per_tile):
        pltpu.sync_copy(ivals.at[j], stamp.at[src_idx.at[j]])

    def gather_round():
      for j in range(rows_per_tile):
        pltpu.sync_copy(stamp.at[idxc.at[j]], got.at[j])

    def recompute_active():
      def grp(g, cnt):
        j = g // (row_w // _L)
        cpos = (g % (row_w // _L)) * _L
        iv = ivals[j, pl.ds(cpos, _L)]
        gv = got[j, pl.ds(cpos, _L)]
        xv = idxc[j, pl.ds(cpos, _L)]
        act = iv > gv
        scat[j, pl.ds(cpos, _L)] = jnp.where(act, xv, dump)
        ones = jnp.where(act, jnp.int32(1), jnp.int32(0))
        return cnt + jnp.sum(ones)
      return lax.fori_loop(0, chunk // _L, grp, jnp.int32(0))

    def exchange_counts(cnt):
      cvec[...] = jnp.full((_L,), cnt, jnp.int32)
      pltpu.sync_copy(cvec, counts_sh.at[s])
      plsc.subcore_barrier()
      pltpu.sync_copy(counts_sh, cbuf)
      acc = lax.fori_loop(
          0, _NS, lambda r, a: a + cbuf[r, :], jnp.zeros((_L,), jnp.int32))
      return jnp.sum(acc)

    scatter_round(idxc)
    plsc.subcore_barrier()
    gather_round()
    total = exchange_counts(recompute_active())

    def body(total):
      scatter_round(scat)
      plsc.subcore_barrier()
      gather_round()
      return exchange_counts(recompute_active())

    lax.while_loop(lambda t: t > 0, body, total)

    gets = [
        pltpu.async_copy(val_hbm.at[got.at[j]],
                         rows.at[pl.ds(j * row_w, row_w)], sem)
        for j in range(rows_per_tile)
    ]
    for g in gets:
      g.wait()

    @pl.when(c == 0)
    def _():
      pltpu.sync_copy(rows, rows_hbm.at[pl.ds(s * chunk, chunk)])

  return winner


def _make_apply(M, B, D):
  n_w = 2 * _NS
  rows_per_w = 128 // n_w
  row_w = B // 128
  chunk = rows_per_w * row_w

  mesh = plsc.VectorSubcoreMesh(core_axis_name="c", subcore_axis_name="s")

  @functools.partial(
      pl.kernel,
      out_type=(),
      mesh=mesh,
      compiler_params=pltpu.CompilerParams(
          needs_layout_passes=False, use_tc_tiling_on_sc=False),
      scratch_types=[
          pltpu.VMEM((rows_per_w, row_w), jnp.int32),
          pltpu.VMEM((rows_per_w, row_w), jnp.int32),
          pltpu.VMEM((chunk, D), jnp.float32),
          pltpu.SemaphoreType.DMA,
      ],
  )
  def apply(rows_hbm, idx_hbm, out_hbm, idxc, q_idx, rows, sem):
    c = lax.axis_index("c")
    s = lax.axis_index("s")
    wid = s * 2 + c
    t0 = wid * rows_per_w

    pltpu.sync_copy(idx_hbm.at[pl.ds(t0, rows_per_w)], idxc)
    pltpu.sync_copy(rows_hbm.at[pl.ds(wid * chunk, chunk)], rows)

    def grp(g, _):
      j = g // (row_w // _L)
      cpos = (g % (row_w // _L)) * _L
      r = idxc[j, pl.ds(cpos, _L)]
      rg = r % _G
      q = 4 * (_P * (r // _G) + r % _P) + rg // _P
      q_idx[j, pl.ds(cpos, _L)] = q
      return 0
    lax.fori_loop(0, chunk // _L, grp, 0)

    puts = [
        pltpu.async_copy(rows.at[pl.ds(j * row_w, row_w)],
                         out_hbm.at[q_idx.at[j]], sem)
        for j in range(rows_per_w)
    ]
    for p in puts:
      p.wait()

  return apply


def kernel(mem, val, idx):
  M, D = mem.shape
  B = val.shape[0]

  idx2d = idx.reshape(128, B // 128)
  biota = jnp.arange(B, dtype=jnp.int32).reshape(128, B // 128)

  rows = _make_winner(M, B, D)(val, idx2d, biota)

  memT = mem.T
  tailT = jnp.pad(memT[:, _TAIL0:], ((0, 0), (0, _P - (M - _TAIL0))))
  rm = _transpose_in(memT, tailT)

  out_ref = jax.new_ref(rm.reshape(_WR * 4, D))
  _make_apply(M, B, D)(rows, idx2d, out_ref)

  outP = _transpose_out(out_ref[...].reshape(_WR, 128), M)
  return outP.T


# speedup vs baseline: 10.0347x; 1.0087x over previous
import functools

import jax
import jax.numpy as jnp
from jax import lax
from jax.experimental import pallas as pl
from jax.experimental.pallas import tpu as pltpu, tpu_sc as plsc

_NS = 16
_L = 16
_G = 65536
_P = _G // 4
_NB = 16
_TAIL0 = 999424
_NFULL_LAST = _TAIL0 // _P - 4 * (_NB - 1)
_CLAMP = _TAIL0 // _P - 1
_WR = _NB * _P


def _transpose_in(memT, tailT):
  def body(x0, x1, x2, x3, t_ref, o_ref):
    i = pl.program_id(0)
    xs = (x0, x1, x2, x3)

    @pl.when(i < _NB - 1)
    def _():
      xcat = jnp.concatenate([x[...] for x in xs], axis=0)
      o_ref[...] = xcat.T

    @pl.when(i == _NB - 1)
    def _():
      pieces = [xs[u][...] if u < _NFULL_LAST else t_ref[...]
                for u in range(4)]
      o_ref[...] = jnp.concatenate(pieces, axis=0).T

  in_spec = [
      pl.BlockSpec((32, _P),
                   (lambda i, u=u: (0, jnp.minimum(4 * i + u, _CLAMP))))
      for u in range(4)
  ]
  return pl.pallas_call(
      body,
      out_shape=jax.ShapeDtypeStruct((_WR, 128), memT.dtype),
      grid=(_NB,),
      in_specs=in_spec + [pl.BlockSpec((32, _P), lambda i: (0, 0))],
      out_specs=pl.BlockSpec((_P, 128), lambda i: (i, 0)),
  )(memT, memT, memT, memT, tailT)


def _transpose_out(rm, M):
  def body(x_ref, o_ref):
    xt = x_ref[...].T
    for u in range(4):
      o_ref[:, _P * u:_P * u + _P] = xt[32 * u:32 * u + 32, :]

  return pl.pallas_call(
      body,
      out_shape=jax.ShapeDtypeStruct((32, M), rm.dtype),
      grid=(_NB,),
      in_specs=[pl.BlockSpec((_P, 128), lambda i: (i, 0))],
      out_specs=pl.BlockSpec((32, _G), lambda i: (0, i)),
  )(rm)


def _make_winner(M, B, D):
  rows_per_tile = 128 // _NS
  row_w = B // 128
  chunk = rows_per_tile * row_w

  mesh = plsc.VectorSubcoreMesh(core_axis_name="c", subcore_axis_name="s")

  @functools.partial(
      pl.kernel,
      out_type=jax.ShapeDtypeStruct((B, D), jnp.float32),
      mesh=mesh,
      compiler_params=pltpu.CompilerParams(
          needs_layout_passes=False, use_tc_tiling_on_sc=False),
      scratch_types=[
          pltpu.VMEM_SHARED((M + 512,), jnp.int32),
          pltpu.VMEM_SHARED((_NS, _L), jnp.int32),
          pltpu.VMEM((rows_per_tile, row_w), jnp.int32),
          pltpu.VMEM((rows_per_tile, row_w), jnp.int32),
          pltpu.VMEM((rows_per_tile, row_w), jnp.int32),
          pltpu.VMEM((rows_per_tile, row_w), jnp.int32),
          pltpu.VMEM((chunk, D), jnp.float32),
          pltpu.VMEM((_L,), jnp.int32),
          pltpu.VMEM((_NS, _L), jnp.int32),
          pltpu.SemaphoreType.DMA,
      ],
  )
  def winner(val_hbm, idx_hbm, iota_hbm, rows_hbm,
             stamp, counts_sh, idxc, ivals, got, scat, rows, cvec, cbuf, sem):
    c = lax.axis_index("c")
    s = lax.axis_index("s")
    t0 = s * rows_per_tile
    lanes = lax.iota(jnp.int32, _L)
    dump = M + s * _L + lanes

    pltpu.sync_copy(idx_hbm.at[pl.ds(t0, rows_per_tile)], idxc)
    pltpu.sync_copy(iota_hbm.at[pl.ds(t0, rows_per_tile)], ivals)

    def scatter_round(src_idx):
      for j in range(rows_per_tile):
        pltpu.sync_copy(ivals.at[j], stamp.at[src_idx.at[j]])

    def gather_round():
      for j in range(rows_per_tile):
        pltpu.sync_copy(stamp.at[idxc.at[j]], got.at[j])

    def recompute_active():
      def grp(g, cnt):
        j = g // (row_w // _L)
        cpos = (g % (row_w // _L)) * _L
        iv = ivals[j, pl.ds(cpos, _L)]
        gv = got[j, pl.ds(cpos, _L)]
        xv = idxc[j, pl.ds(cpos, _L)]
        act = iv > gv
        scat[j, pl.ds(cpos, _L)] = jnp.where(act, xv, dump)
        ones = jnp.where(act, jnp.int32(1), jnp.int32(0))
        return cnt + jnp.sum(ones)
      return lax.fori_loop(0, chunk // _L, grp, jnp.int32(0))

    def exchange_counts(cnt):
      cvec[...] = jnp.full((_L,), cnt, jnp.int32)
      pltpu.sync_copy(cvec, counts_sh.at[s])
      plsc.subcore_barrier()
      pltpu.sync_copy(counts_sh, cbuf)
      acc = lax.fori_loop(
          0, _NS, lambda r, a: a + cbuf[r, :], jnp.zeros((_L,), jnp.int32))
      return jnp.sum(acc)

    scatter_round(idxc)
    plsc.subcore_barrier()
    gather_round()
    total = exchange_counts(recompute_active())

    def body(total):
      scatter_round(scat)
      plsc.subcore_barrier()
      gather_round()
      return exchange_counts(recompute_active())

    lax.while_loop(lambda t: t > 0, body, total)

    gets = [
        pltpu.async_copy(val_hbm.at[got.at[j]],
                         rows.at[pl.ds(j * row_w, row_w)], sem)
        for j in range(rows_per_tile)
    ]
    for g in gets:
      g.wait()

    @pl.when(c == 0)
    def _():
      pltpu.sync_copy(rows, rows_hbm.at[pl.ds(s * chunk, chunk)])

  return winner


def _make_apply(M, B, D):
  n_w = 2 * _NS
  rows_per_w = 128 // n_w
  row_w = B // 128
  chunk = rows_per_w * row_w

  mesh = plsc.VectorSubcoreMesh(core_axis_name="c", subcore_axis_name="s")

  @functools.partial(
      pl.kernel,
      out_type=(),
      mesh=mesh,
      compiler_params=pltpu.CompilerParams(
          needs_layout_passes=False, use_tc_tiling_on_sc=False),
      scratch_types=[
          pltpu.VMEM((rows_per_w, row_w), jnp.int32),
          pltpu.VMEM((rows_per_w, row_w), jnp.int32),
          pltpu.VMEM((chunk, D), jnp.float32),
          pltpu.SemaphoreType.DMA,
      ],
  )
  def apply(rows_hbm, idx_hbm, out_hbm, idxc, q_idx, rows, sem):
    c = lax.axis_index("c")
    s = lax.axis_index("s")
    wid = s * 2 + c
    t0 = wid * rows_per_w

    pltpu.sync_copy(idx_hbm.at[pl.ds(t0, rows_per_w)], idxc)
    pltpu.sync_copy(rows_hbm.at[pl.ds(wid * chunk, chunk)], rows)

    def grp(g, _):
      j = g // (row_w // _L)
      cpos = (g % (row_w // _L)) * _L
      r = idxc[j, pl.ds(cpos, _L)]
      rg = r % _G
      q = 4 * (_P * (r // _G) + r % _P) + rg // _P
      q_idx[j, pl.ds(cpos, _L)] = q
      return 0
    lax.fori_loop(0, chunk // _L, grp, 0)

    puts = [
        pltpu.async_copy(rows.at[pl.ds(j * row_w, row_w)],
                         out_hbm.at[q_idx.at[j]], sem)
        for j in range(rows_per_w)
    ]
    for p in puts:
      p.wait()

  return apply


def kernel(mem, val, idx):
  M, D = mem.shape
  B = val.shape[0]

  idx2d = idx.reshape(128, B // 128)
  biota = jnp.arange(B, dtype=jnp.int32).reshape(128, B // 128)

  rows = _make_winner(M, B, D)(val, idx2d, biota)

  memT = mem.T
  tailT = jnp.pad(memT[:, _TAIL0:], ((0, 0), (0, _P - (M - _TAIL0))))
  rm = _transpose_in(memT, tailT)

  out_ref = jax.new_ref(rm.reshape(_WR * 4, D))
  _make_apply(M, B, D)(rows, idx2d, out_ref)

  outP = _transpose_out(out_ref[...].reshape(_WR, 128), M)
  return outP.T
